# Initial kernel scaffold; baseline (speedup 1.0000x reference)
import jax
import jax.numpy as jnp
from jax.experimental import pallas as pl


def kernel(x, position_embeddings):
    raise NotImplementedError("write your pallas kernel here")


# speedup vs baseline: 8.6827x; 8.6827x over previous
import functools

import jax
import jax.numpy as jnp
from jax import lax
from jax.experimental import pallas as pl
from jax.experimental.pallas import tpu as pltpu
from jax.experimental.pallas import tpu_sc as plsc

MAX_LEN = 32768
DIM = 128
N_IDX = 262144

_NC = 2
_NS = 16
_NW = _NC * _NS
_BPW = N_IDX // _NW
_CHUNK = 128
_NCHUNK = _BPW // _CHUNK


@functools.partial(
    pl.kernel,
    mesh=plsc.VectorSubcoreMesh(core_axis_name="c", subcore_axis_name="s"),
    out_type=jax.ShapeDtypeStruct((N_IDX, DIM), jnp.float32),
    scratch_types=[
        pltpu.VMEM((_BPW,), jnp.int32),
        pltpu.VMEM((_CHUNK, DIM), jnp.float32),
        pltpu.VMEM((_CHUNK, DIM), jnp.float32),
        pltpu.SemaphoreType.DMA,
        pltpu.SemaphoreType.DMA,
    ],
)
def _emb(table_hbm, idx_hbm, out_hbm, idx_v, rows0, rows1, gsem0, gsem1):
    wid = lax.axis_index("s") * _NC + lax.axis_index("c")
    base = wid * _BPW

    pltpu.sync_copy(idx_hbm.at[pl.ds(base, _BPW)], idx_v)

    rows = (rows0, rows1)
    gsems = (gsem0, gsem1)

    def start_gather(b, j):
        pltpu.async_copy(
            table_hbm.at[idx_v.at[pl.ds(j * _CHUNK, _CHUNK)]],
            rows[b], gsems[b])

    def wait_gather(b, j):
        pltpu.make_async_copy(
            table_hbm.at[idx_v.at[pl.ds(j * _CHUNK, _CHUNK)]],
            rows[b], gsems[b]).wait()

    start_gather(0, 0)

    def outer(j0, carry):
        for b in range(2):
            j = j0 * 2 + b
            nxt = j + 1

            @pl.when(nxt < _NCHUNK)
            def _():
                start_gather(1 - b, nxt)

            wait_gather(b, j)
            pltpu.sync_copy(
                rows[b], out_hbm.at[pl.ds(base + j * _CHUNK, _CHUNK)])
        return carry

    lax.fori_loop(0, _NCHUNK // 2, outer, 0)


def kernel(x, position_embeddings):
    return _emb(position_embeddings, x)


# speedup vs baseline: 8.8299x; 1.0170x over previous
import functools

import jax
import jax.numpy as jnp
from jax import lax
from jax.experimental import pallas as pl
from jax.experimental.pallas import tpu as pltpu
from jax.experimental.pallas import tpu_sc as plsc

MAX_LEN = 32768
DIM = 128
N_IDX = 262144

_NC = 2
_NS = 16
_NW = _NC * _NS
_BPW = N_IDX // _NW
_CHUNK = 256
_NCHUNK = _BPW // _CHUNK


@functools.partial(
    pl.kernel,
    mesh=plsc.VectorSubcoreMesh(core_axis_name="c", subcore_axis_name="s"),
    out_type=jax.ShapeDtypeStruct((N_IDX, DIM), jnp.float32),
    scratch_types=[
        pltpu.VMEM((_BPW,), jnp.int32),
        pltpu.VMEM((_CHUNK, DIM), jnp.float32),
        pltpu.VMEM((_CHUNK, DIM), jnp.float32),
        pltpu.SemaphoreType.DMA,
        pltpu.SemaphoreType.DMA,
    ],
)
def _emb(table_hbm, idx_hbm, out_hbm, idx_v, rows0, rows1, gsem0, gsem1):
    wid = lax.axis_index("s") * _NC + lax.axis_index("c")
    base = wid * _BPW

    pltpu.sync_copy(idx_hbm.at[pl.ds(base, _BPW)], idx_v)

    rows = (rows0, rows1)
    gsems = (gsem0, gsem1)

    def start_gather(b, j):
        pltpu.async_copy(
            table_hbm.at[idx_v.at[pl.ds(j * _CHUNK, _CHUNK)]],
            rows[b], gsems[b])

    def wait_gather(b, j):
        pltpu.make_async_copy(
            table_hbm.at[idx_v.at[pl.ds(j * _CHUNK, _CHUNK)]],
            rows[b], gsems[b]).wait()

    start_gather(0, 0)

    def outer(j0, carry):
        for b in range(2):
            j = j0 * 2 + b
            nxt = j + 1

            @pl.when(nxt < _NCHUNK)
            def _():
                start_gather(1 - b, nxt)

            wait_gather(b, j)
            pltpu.sync_copy(
                rows[b], out_hbm.at[pl.ds(base + j * _CHUNK, _CHUNK)])
        return carry

    lax.fori_loop(0, _NCHUNK // 2, outer, 0)


def kernel(x, position_embeddings):
    return _emb(position_embeddings, x)


# speedup vs baseline: 8.8339x; 1.0004x over previous
import functools

import jax
import jax.numpy as jnp
from jax import lax
from jax.experimental import pallas as pl
from jax.experimental.pallas import tpu as pltpu
from jax.experimental.pallas import tpu_sc as plsc

MAX_LEN = 32768
DIM = 128
N_IDX = 262144

_NC = 2
_NS = 16
_NW = _NC * _NS
_BPW = N_IDX // _NW
_CHUNK = 128
_NCHUNK = _BPW // _CHUNK
_NBUF = 4


@functools.partial(
    pl.kernel,
    mesh=plsc.VectorSubcoreMesh(core_axis_name="c", subcore_axis_name="s"),
    out_type=jax.ShapeDtypeStruct((N_IDX, DIM), jnp.float32),
    scratch_types=[
        pltpu.VMEM((_BPW,), jnp.int32),
        pltpu.VMEM((_CHUNK, DIM), jnp.float32),
        pltpu.VMEM((_CHUNK, DIM), jnp.float32),
        pltpu.VMEM((_CHUNK, DIM), jnp.float32),
        pltpu.VMEM((_CHUNK, DIM), jnp.float32),
        pltpu.SemaphoreType.DMA,
        pltpu.SemaphoreType.DMA,
        pltpu.SemaphoreType.DMA,
        pltpu.SemaphoreType.DMA,
        pltpu.SemaphoreType.DMA,
        pltpu.SemaphoreType.DMA,
        pltpu.SemaphoreType.DMA,
        pltpu.SemaphoreType.DMA,
    ],
)
def _emb(table_hbm, idx_hbm, out_hbm, idx_v,
         rows0, rows1, rows2, rows3,
         g0, g1, g2, g3, o0, o1, o2, o3):
    wid = lax.axis_index("s") * _NC + lax.axis_index("c")
    base = wid * _BPW

    pltpu.sync_copy(idx_hbm.at[pl.ds(base, _BPW)], idx_v)

    rows = (rows0, rows1, rows2, rows3)
    gsem = (g0, g1, g2, g3)
    osem = (o0, o1, o2, o3)

    def start_g(b, j):
        pltpu.async_copy(
            table_hbm.at[idx_v.at[pl.ds(j * _CHUNK, _CHUNK)]],
            rows[b], gsem[b])

    def wait_g(b, j):
        pltpu.make_async_copy(
            table_hbm.at[idx_v.at[pl.ds(j * _CHUNK, _CHUNK)]],
            rows[b], gsem[b]).wait()

    def start_o(b, j):
        pltpu.async_copy(
            rows[b], out_hbm.at[pl.ds(base + j * _CHUNK, _CHUNK)], osem[b])

    def wait_o(b, j):
        pltpu.make_async_copy(
            rows[b], out_hbm.at[pl.ds(base + j * _CHUNK, _CHUNK)],
            osem[b]).wait()

    start_g(0, 0)
    start_g(1, 1)
    start_g(2, 2)
    wait_g(0, 0)
    start_o(0, 0)
    start_g(3, 3)
    wait_g(1, 1)
    start_o(1, 1)

    def outer(j0, carry):
        for b2 in range(_NBUF):
            j = 2 + j0 * _NBUF + b2
            b = (2 + b2) % _NBUF
            bpre = b2
            wait_o(bpre, j - 2)
            start_g(bpre, j + 2)
            wait_g(b, j)
            start_o(b, j)
        return carry

    lax.fori_loop(0, (_NCHUNK - 4) // _NBUF, outer, 0)

    for j in (_NCHUNK - 2, _NCHUNK - 1):
        b = j % _NBUF
        wait_o((j + 2) % _NBUF, j - 2)
        wait_g(b, j)
        start_o(b, j)
    wait_o((_NCHUNK - 2) % _NBUF, _NCHUNK - 2)
    wait_o((_NCHUNK - 1) % _NBUF, _NCHUNK - 1)


def kernel(x, position_embeddings):
    return _emb(position_embeddings, x)
